# baseline (speedup 1.0000x reference)
import functools
import math

import jax
import jax.numpy as jnp
import numpy as np
from jax import lax
from jax.experimental import pallas as pl
from jax.experimental.pallas import tpu as pltpu
from jax.experimental.pallas import tpu_sc as plsc

N = 10000
E = 320000
D = 128
DE = 16
H = 8
C = 16
FFN = 512
EPS = 1e-8

PAY = 144
NC, NS = 2, 16
NW = NC * NS
EPT = E // NW
CH = 16
NG = EPT // CH
ROWS_PT = N // NS
SQRT_D = math.sqrt(D)
INV_SQRT_C = 1.0 / math.sqrt(C)

BN = 400
BE = 3200

_ILV = plsc.PackFormat.INTERLEAVED

_PERM_T = np.empty(2 * D, np.int32)
for _h in range(H):
    for _k in range(C):
        _PERM_T[32 * _h + 2 * _k] = 16 * _h + _k
        _PERM_T[32 * _h + 2 * _k + 1] = D + 16 * _h + _k
_PERM_V = np.empty(D, np.int32)
for _j in range(H // 2):
    for _k in range(C):
        _PERM_V[32 * _j + 2 * _k] = 32 * _j + _k
        _PERM_V[32 * _j + 2 * _k + 1] = 32 * _j + 16 + _k


def _bcast_lane(v, h):
    idx = jnp.full((16,), h, jnp.int32)
    return v.at[idx].get(mode="promise_in_bounds")


_HIMASK = np.int32(-65536)


def _f32lo(w):
    return jax.lax.bitcast_convert_type(jax.lax.shift_left(w, 16), jnp.float32)


def _f32hi(w):
    return jax.lax.bitcast_convert_type(jnp.bitwise_and(w, _HIMASK),
                                        jnp.float32)


def _pre_body(nf, s_attn, wcat_t, bcat_t, wcat_s, t_tgt, t_src):
    x = nf[...]
    nrm = jnp.sqrt(jnp.sum(x * x, axis=1, keepdims=True))
    h = s_attn[...] * x / (nrm / SQRT_D + EPS)
    t_tgt[...] = (jnp.dot(h, wcat_t[...], preferred_element_type=jnp.float32)
                  + bcat_t[...]).astype(jnp.bfloat16)
    t_src[...] = jnp.dot(h, wcat_s[...],
                         preferred_element_type=jnp.float32).astype(jnp.bfloat16)


def _ve_body(ef, wve, bv, ve):
    ve[...] = (jnp.dot(ef[...], wve[...], preferred_element_type=jnp.float32)
               + bv[...]).astype(jnp.bfloat16)


def _post_body(acc, nf, wo, bo, srep, s_ffn, w1, w2, out):
    a = acc[0] + acc[1]
    arep = jnp.dot(a, srep[...], preferred_element_type=jnp.float32)
    attn = a[:, :D] * (1.0 / (arep + 1e-16))
    y = jnp.dot(attn, wo[...], preferred_element_type=jnp.float32) + bo[...]
    x1 = nf[...] + y
    nrm = jnp.sqrt(jnp.sum(x1 * x1, axis=1, keepdims=True))
    h2 = s_ffn[...] * x1 / (nrm / SQRT_D + EPS)
    g = jax.nn.gelu(jnp.dot(h2, w1[...], preferred_element_type=jnp.float32))
    out[...] = x1 + jnp.dot(g, w2[...], preferred_element_type=jnp.float32)


def _sc_edge_body(t_tgt, t_src, ve_hbm, ef_hbm, src_hbm, tgt_hbm, out_hbm,
                  sidx0, sidx1, tidx0, tidx1, ve0, ve1, ef0, ef1,
                  rt0, rt1, rs0, rs1, pay, acc,
                  s_si0, s_si1, s_ti0, s_ti1, s_ve0, s_ve1, s_ef0, s_ef1,
                  s_rt0, s_rt1, s_rs0, s_rs1):
    c = lax.axis_index("c")
    s = lax.axis_index("s")
    wid = c * NS + s
    zero16 = jnp.zeros((16,), jnp.float32)
    lane = lax.iota(jnp.int32, 16)
    ivef = ((sidx0, tidx0, ve0, ef0, s_si0, s_ti0, s_ve0, s_ef0),
            (sidx1, tidx1, ve1, ef1, s_si1, s_ti1, s_ve1, s_ef1))
    rows = ((rt0, rs0, s_rt0, s_rs0), (rt1, rs1, s_rt1, s_rs1))

    def zrow(i, carry):
        for j in range(PAY // 16):
            pay[i, pl.ds(j * 16, 16)] = zero16
        return carry

    lax.fori_loop(0, CH, zrow, None)
    rowbase = s * ROWS_PT
    nfull = ROWS_PT // CH
    rem = ROWS_PT - nfull * CH
    for j in range(nfull):
        pltpu.sync_copy(pay.at[pl.ds(0, CH)],
                        acc.at[pl.ds(rowbase + j * CH, CH)])
    if rem:
        pltpu.sync_copy(pay.at[pl.ds(0, rem)],
                        acc.at[pl.ds(rowbase + nfull * CH, rem)])
    plsc.subcore_barrier()

    ebase = wid * EPT

    def issue_ivef(off, b):
        si, ti, ve, ef, ssi, sti, sve, sef = ivef[b]
        pltpu.async_copy(src_hbm.at[pl.ds(off, CH)], si, ssi)
        pltpu.async_copy(tgt_hbm.at[pl.ds(off, CH)], ti, sti)
        pltpu.async_copy(ve_hbm.at[pl.ds(off, CH)], ve, sve)
        pltpu.async_copy(ef_hbm.at[pl.ds(off, CH)], ef, sef)

    def wait_idx(off, b):
        si, ti, _, _, ssi, sti, _, _ = ivef[b]
        pltpu.make_async_copy(src_hbm.at[pl.ds(off, CH)], si, ssi).wait()
        pltpu.make_async_copy(tgt_hbm.at[pl.ds(off, CH)], ti, sti).wait()

    def issue_rows(b):
        si, ti = ivef[b][0], ivef[b][1]
        rt, rs, srt, srs = rows[b]
        pltpu.async_copy(t_tgt.at[ti], rt, srt)
        pltpu.async_copy(t_src.at[si], rs, srs)

    def wait_rows(b):
        si, ti = ivef[b][0], ivef[b][1]
        rt, rs, srt, srs = rows[b]
        pltpu.make_async_copy(t_tgt.at[ti], rt, srt).wait()
        pltpu.make_async_copy(t_src.at[si], rs, srs).wait()

    def compute(b, off):
        _, _, ve_v, ef_v, _, _, sve, sef = ivef[b]
        rt, rs, _, _ = rows[b]
        pltpu.make_async_copy(ve_hbm.at[pl.ds(off, CH)], ve_v, sve).wait()
        pltpu.make_async_copy(ef_hbm.at[pl.ds(off, CH)], ef_v, sef).wait()
        wait_rows(b)
        efm = [plsc.load_gather(ef_v, [lane, jnp.full((16,), m, jnp.int32)])
               for m in range(C)]
        pv = []
        for h in range(H):
            acc_h = zero16
            for m in range(C):
                col = jnp.full((16,), 16 * h + m, jnp.int32)
                wt = plsc.load_gather(rt, [lane, col])
                ws = plsc.load_gather(rs, [lane, col])
                acc_h = acc_h + _f32lo(wt) * _f32lo(ws) + _f32hi(wt) * efm[m]
            pv.append(jnp.exp(acc_h))
        for h in range(H):
            plsc.store_scatter(pay, [lane, jnp.full((16,), 128 + h, jnp.int32)],
                               pv[h])
        for i in range(CH):
            for j in range(H // 2):
                wv = ve_v[i, pl.ds(16 * j, 16)]
                vea = _f32lo(wv)
                veb = _f32hi(wv)
                for p, veh in ((0, vea), (1, veb)):
                    h = 2 * j + p
                    wn = rs[i, pl.ds(16 * h, 16)]
                    vn = _f32hi(wn)
                    pb = _bcast_lane(pv[h], i)
                    pay[i, pl.ds(16 * h, 16)] = pb * (vn + veh)

    def body(g, b):
        off = ebase + g * CH

        @pl.when(g + 1 < NG)
        def _():
            wait_idx(off + CH, 1 - b)
            issue_rows(1 - b)

        @pl.when(g < NG)
        def _():
            compute(b, off)
            pltpu.sync_copy(pay, acc.at[ivef[b][1]], add=True)

        @pl.when(g + 2 < NG)
        def _():
            issue_ivef(off + 2 * CH, b)

    issue_ivef(ebase, 0)
    wait_idx(ebase, 0)
    issue_rows(0)
    issue_ivef(ebase + CH, 1)

    def pair(gp, carry):
        body(2 * gp, 0)
        body(2 * gp + 1, 1)
        return carry

    lax.fori_loop(0, (NG + 2) // 2, pair, None)
    plsc.subcore_barrier()
    for j in range(nfull):
        pltpu.sync_copy(acc.at[pl.ds(rowbase + j * CH, CH)],
                        out_hbm.at[c, pl.ds(rowbase + j * CH, CH)])
    if rem:
        pltpu.sync_copy(acc.at[pl.ds(rowbase + nfull * CH, rem)],
                        out_hbm.at[c, pl.ds(rowbase + nfull * CH, rem)])


def _pre_call(nf, s_attn, wcat_t, bcat_t, wcat_s):
    grid = N // BN
    return pl.pallas_call(
        _pre_body,
        grid=(grid,),
        in_specs=[
            pl.BlockSpec((BN, D), lambda i: (i, 0)),
            pl.BlockSpec((1, D), lambda i: (0, 0)),
            pl.BlockSpec((D, 2 * D), lambda i: (0, 0)),
            pl.BlockSpec((1, 2 * D), lambda i: (0, 0)),
            pl.BlockSpec((D, 2 * D), lambda i: (0, 0)),
        ],
        out_specs=[
            pl.BlockSpec((BN, 2 * D), lambda i: (i, 0)),
            pl.BlockSpec((BN, 2 * D), lambda i: (i, 0)),
        ],
        out_shape=[
            jax.ShapeDtypeStruct((N, 2 * D), jnp.bfloat16),
            jax.ShapeDtypeStruct((N, 2 * D), jnp.bfloat16),
        ],
    )(nf, s_attn, wcat_t, bcat_t, wcat_s)


def _ve_call(ef, wve, bv):
    grid = E // BE
    return pl.pallas_call(
        _ve_body,
        grid=(grid,),
        in_specs=[
            pl.BlockSpec((BE, DE), lambda i: (i, 0)),
            pl.BlockSpec((DE, D), lambda i: (0, 0)),
            pl.BlockSpec((1, D), lambda i: (0, 0)),
        ],
        out_specs=pl.BlockSpec((BE, D), lambda i: (i, 0)),
        out_shape=jax.ShapeDtypeStruct((E, D), jnp.bfloat16),
    )(ef, wve, bv)


def _post_call(acc, nf, wo, bo, srep, s_ffn, w1, w2):
    grid = N // BN
    return pl.pallas_call(
        _post_body,
        grid=(grid,),
        in_specs=[
            pl.BlockSpec((2, BN, PAY), lambda i: (0, i, 0)),
            pl.BlockSpec((BN, D), lambda i: (i, 0)),
            pl.BlockSpec((D, D), lambda i: (0, 0)),
            pl.BlockSpec((1, D), lambda i: (0, 0)),
            pl.BlockSpec((PAY, D), lambda i: (0, 0)),
            pl.BlockSpec((1, D), lambda i: (0, 0)),
            pl.BlockSpec((D, FFN), lambda i: (0, 0)),
            pl.BlockSpec((FFN, D), lambda i: (0, 0)),
        ],
        out_specs=pl.BlockSpec((BN, D), lambda i: (i, 0)),
        out_shape=jax.ShapeDtypeStruct((N, D), jnp.float32),
    )(acc, nf, wo, bo, srep, s_ffn, w1, w2)


_sc_edge_call = functools.partial(
    pl.kernel,
    out_type=jax.ShapeDtypeStruct((NC, N, PAY), jnp.float32),
    mesh=plsc.VectorSubcoreMesh(core_axis_name="c", subcore_axis_name="s"),
    compiler_params=pltpu.CompilerParams(use_tc_tiling_on_sc=False,
                                         needs_layout_passes=False),
    scratch_types=(
        [pltpu.VMEM((CH,), jnp.int32)] * 4
        + [pltpu.VMEM((CH, D // 2), jnp.int32)] * 2
        + [pltpu.VMEM((CH, DE), jnp.float32)] * 2
        + [pltpu.VMEM((CH, D), jnp.int32)] * 4
        + [pltpu.VMEM((CH, PAY), jnp.float32)]
        + [pltpu.VMEM_SHARED((N, PAY), jnp.float32)]
        + [pltpu.SemaphoreType.DMA] * 12
    ),
)(_sc_edge_body)


def kernel(node_feats, edge_feats, edge_index, Wq, bq, Wk, bk, Wv, bv,
           Wo, bo, s_attn, s_ffn, W1, W2):
    src = edge_index[0]
    tgt = edge_index[1]
    we = Wk[D:].reshape(DE, H, C)
    wblk = jnp.einsum('jhc,hg->hcgj', we, jnp.eye(H, dtype=jnp.float32))
    wblk = wblk.reshape(H * C, H * DE)
    wq_s = Wq * INV_SQRT_C
    bq_s = bq * INV_SQRT_C
    wcat_t = jnp.concatenate([wq_s, wq_s @ wblk], axis=1)[:, _PERM_T]
    bcat_t = jnp.concatenate([bq_s, bq_s @ wblk])[_PERM_T].reshape(1, 2 * D)
    wcat_s = jnp.concatenate([Wk[:D], Wv[:D]], axis=1)[:, _PERM_T]
    wve = Wv[D:][:, _PERM_V]
    bv_p = bv[_PERM_V].reshape(1, D)
    srep = jnp.concatenate(
        [jnp.zeros((D, D), jnp.float32),
         jnp.kron(jnp.eye(H, dtype=jnp.float32), jnp.ones((1, C), jnp.float32)),
         jnp.zeros((PAY - D - H, D), jnp.float32)], axis=0)

    t_tgt, t_src = _pre_call(node_feats, s_attn.reshape(1, D),
                             wcat_t, bcat_t, wcat_s)
    ve = _ve_call(edge_feats, wve, bv_p)
    t_tgt_i = lax.bitcast_convert_type(t_tgt.reshape(N, D, 2), jnp.int32)
    t_src_i = lax.bitcast_convert_type(t_src.reshape(N, D, 2), jnp.int32)
    ve_i = lax.bitcast_convert_type(ve.reshape(E, D // 2, 2), jnp.int32)
    acc = _sc_edge_call(t_tgt_i, t_src_i, ve_i, edge_feats, src, tgt)
    out = _post_call(acc, node_feats, Wo, bo.reshape(1, D), srep,
                     s_ffn.reshape(1, D), W1, W2)
    return out


# speedup vs baseline: 1.4633x; 1.4633x over previous
import functools
import math

import jax
import jax.numpy as jnp
import numpy as np
from jax import lax
from jax.experimental import pallas as pl
from jax.experimental.pallas import tpu as pltpu
from jax.experimental.pallas import tpu_sc as plsc

N = 10000
E = 320000
D = 128
DE = 16
H = 8
C = 16
FFN = 512
EPS = 1e-8

PAY = 144
NC, NS = 2, 16
NW = NC * NS
EPT = E // NW
CH = 40
NG = EPT // CH
ROWS_PT = N // NS
SQRT_D = math.sqrt(D)
INV_SQRT_C = 1.0 / math.sqrt(C)

BN = 400
BE = 3200

_ILV = plsc.PackFormat.INTERLEAVED

_PERM_T = np.empty(2 * D, np.int32)
for _h in range(H):
    for _k in range(C):
        _PERM_T[32 * _h + 2 * _k] = 16 * _h + _k
        _PERM_T[32 * _h + 2 * _k + 1] = D + 16 * _h + _k
_PERM_V = np.empty(D, np.int32)
for _j in range(H // 2):
    for _k in range(C):
        _PERM_V[32 * _j + 2 * _k] = 32 * _j + _k
        _PERM_V[32 * _j + 2 * _k + 1] = 32 * _j + 16 + _k


def _permv(v, idx):
    return v.at[idx].get(mode="promise_in_bounds")


_HIMASK = np.int32(-65536)


def _f32lo(w):
    return jax.lax.bitcast_convert_type(jax.lax.shift_left(w, 16), jnp.float32)


def _f32hi(w):
    return jax.lax.bitcast_convert_type(jnp.bitwise_and(w, _HIMASK),
                                        jnp.float32)


def _pre_body(nf, s_attn, wcat_t, bcat_t, wcat_s, t_tgt, t_src):
    x = nf[...]
    nrm = jnp.sqrt(jnp.sum(x * x, axis=1, keepdims=True))
    h = s_attn[...] * x / (nrm / SQRT_D + EPS)
    t_tgt[...] = (jnp.dot(h, wcat_t[...], preferred_element_type=jnp.float32)
                  + bcat_t[...]).astype(jnp.bfloat16)
    t_src[...] = jnp.dot(h, wcat_s[...],
                         preferred_element_type=jnp.float32).astype(jnp.bfloat16)


def _ve_body(ef, wve, bv, ve):
    ve[...] = (jnp.dot(ef[...], wve[...], preferred_element_type=jnp.float32)
               + bv[...]).astype(jnp.bfloat16)


def _post_body(acc, nf, wo, bo, srep, s_ffn, w1, w2, out):
    a = acc[0] + acc[1]
    arep = jnp.dot(a, srep[...], preferred_element_type=jnp.float32)
    attn = a[:, :D] * (1.0 / (arep + 1e-16))
    y = jnp.dot(attn, wo[...], preferred_element_type=jnp.float32) + bo[...]
    x1 = nf[...] + y
    nrm = jnp.sqrt(jnp.sum(x1 * x1, axis=1, keepdims=True))
    h2 = s_ffn[...] * x1 / (nrm / SQRT_D + EPS)
    g = jax.nn.gelu(jnp.dot(h2, w1[...], preferred_element_type=jnp.float32))
    out[...] = x1 + jnp.dot(g, w2[...], preferred_element_type=jnp.float32)


def _sc_edge_body(t_tgt, t_src, ve_hbm, ef_hbm, src_hbm, tgt_hbm, out_hbm,
                  sidx0, sidx1, tidx0, tidx1, ve0, ve1, ef0, ef1,
                  rt0, rt1, rs0, rs1, pay, acc,
                  s_si0, s_si1, s_ti0, s_ti1, s_ve0, s_ve1, s_ef0, s_ef1,
                  s_rt0, s_rt1, s_rs0, s_rs1):
    c = lax.axis_index("c")
    s = lax.axis_index("s")
    wid = c * NS + s
    zero16 = jnp.zeros((16,), jnp.float32)
    lane = lax.iota(jnp.int32, 16)
    ivef = ((sidx0, tidx0, ve0, ef0, s_si0, s_ti0, s_ve0, s_ef0),
            (sidx1, tidx1, ve1, ef1, s_si1, s_ti1, s_ve1, s_ef1))
    rows = ((rt0, rs0, s_rt0, s_rs0), (rt1, rs1, s_rt1, s_rs1))

    def zrow(i, carry):
        for j in range(PAY // 16):
            pay[i, pl.ds(j * 16, 16)] = zero16
        return carry

    lax.fori_loop(0, CH, zrow, None)
    rowbase = s * ROWS_PT
    nfull = ROWS_PT // CH
    rem = ROWS_PT - nfull * CH
    for j in range(nfull):
        pltpu.sync_copy(pay.at[pl.ds(0, CH)],
                        acc.at[pl.ds(rowbase + j * CH, CH)])
    if rem:
        pltpu.sync_copy(pay.at[pl.ds(0, rem)],
                        acc.at[pl.ds(rowbase + nfull * CH, rem)])
    plsc.subcore_barrier()

    ebase = wid * EPT

    def issue_ivef(off, b):
        si, ti, ve, ef, ssi, sti, sve, sef = ivef[b]
        pltpu.async_copy(src_hbm.at[pl.ds(off, CH)], si, ssi)
        pltpu.async_copy(tgt_hbm.at[pl.ds(off, CH)], ti, sti)
        pltpu.async_copy(ve_hbm.at[pl.ds(off, CH)], ve, sve)
        pltpu.async_copy(ef_hbm.at[pl.ds(off, CH)], ef, sef)

    def wait_idx(off, b):
        si, ti, _, _, ssi, sti, _, _ = ivef[b]
        pltpu.make_async_copy(src_hbm.at[pl.ds(off, CH)], si, ssi).wait()
        pltpu.make_async_copy(tgt_hbm.at[pl.ds(off, CH)], ti, sti).wait()

    def issue_rows(b):
        si, ti = ivef[b][0], ivef[b][1]
        rt, rs, srt, srs = rows[b]
        pltpu.async_copy(t_tgt.at[ti], rt, srt)
        pltpu.async_copy(t_src.at[si], rs, srs)

    def wait_rows(b):
        si, ti = ivef[b][0], ivef[b][1]
        rt, rs, srt, srs = rows[b]
        pltpu.make_async_copy(t_tgt.at[ti], rt, srt).wait()
        pltpu.make_async_copy(t_src.at[si], rs, srs).wait()

    def compute(b, off):
        _, _, ve_v, ef_v, _, _, sve, sef = ivef[b]
        rt, rs, _, _ = rows[b]
        pltpu.make_async_copy(ve_hbm.at[pl.ds(off, CH)], ve_v, sve).wait()
        pltpu.make_async_copy(ef_hbm.at[pl.ds(off, CH)], ef_v, sef).wait()
        wait_rows(b)
        x8 = jnp.bitwise_xor(lane, 8)
        x4 = jnp.bitwise_xor(lane, 4)
        x2 = jnp.bitwise_xor(lane, 2)
        x1 = jnp.bitwise_xor(lane, 1)
        lhalf = lane < 8
        idx0 = jnp.full((16,), 0, jnp.int32)
        idx8 = jnp.full((16,), 8, jnp.int32)
        lmask = [lane == h for h in range(H)]

        def hsum2(pa, pb):
            a = pa + _permv(pa, x8)
            bb = pb + _permv(pb, x8)
            m = jnp.where(lhalf, a, bb)
            m = m + _permv(m, x4)
            m = m + _permv(m, x2)
            return m + _permv(m, x1)

        def edge2(k, ecarry):
            for u in range(2):
                i = 2 * k + u
                efe = ef_v[i, :]
                vns = []
                prods = []
                for h in range(H):
                    wt = rt[i, pl.ds(16 * h, 16)]
                    ws = rs[i, pl.ds(16 * h, 16)]
                    vns.append(_f32hi(ws))
                    prods.append(_f32lo(wt) * _f32lo(ws) + _f32hi(wt) * efe)
                es = [jnp.exp(hsum2(prods[2 * j], prods[2 * j + 1]))
                      for j in range(H // 2)]
                pbs = [_permv(es[h // 2], idx0 if h % 2 == 0 else idx8)
                       for h in range(H)]
                s = zero16
                for h in range(H):
                    s = jnp.where(lmask[h], pbs[h], s)
                pay[i, pl.ds(128, 16)] = s
                for j in range(H // 2):
                    wv = ve_v[i, pl.ds(16 * j, 16)]
                    pay[i, pl.ds(32 * j, 16)] = (
                        pbs[2 * j] * (vns[2 * j] + _f32lo(wv)))
                    pay[i, pl.ds(32 * j + 16, 16)] = (
                        pbs[2 * j + 1] * (vns[2 * j + 1] + _f32hi(wv)))
            return ecarry

        lax.fori_loop(0, CH // 2, edge2, None)

    def body(g, b):
        off = ebase + g * CH

        @pl.when(g + 1 < NG)
        def _():
            wait_idx(off + CH, 1 - b)
            issue_rows(1 - b)

        @pl.when(g < NG)
        def _():
            compute(b, off)
            pltpu.sync_copy(pay, acc.at[ivef[b][1]], add=True)

        @pl.when(g + 2 < NG)
        def _():
            issue_ivef(off + 2 * CH, b)

    issue_ivef(ebase, 0)
    wait_idx(ebase, 0)
    issue_rows(0)
    issue_ivef(ebase + CH, 1)

    def pair(gp, carry):
        body(2 * gp, 0)
        body(2 * gp + 1, 1)
        return carry

    lax.fori_loop(0, (NG + 2) // 2, pair, None)
    plsc.subcore_barrier()
    for j in range(nfull):
        pltpu.sync_copy(acc.at[pl.ds(rowbase + j * CH, CH)],
                        out_hbm.at[c, pl.ds(rowbase + j * CH, CH)])
    if rem:
        pltpu.sync_copy(acc.at[pl.ds(rowbase + nfull * CH, rem)],
                        out_hbm.at[c, pl.ds(rowbase + nfull * CH, rem)])


def _pre_call(nf, s_attn, wcat_t, bcat_t, wcat_s):
    grid = N // BN
    return pl.pallas_call(
        _pre_body,
        grid=(grid,),
        in_specs=[
            pl.BlockSpec((BN, D), lambda i: (i, 0)),
            pl.BlockSpec((1, D), lambda i: (0, 0)),
            pl.BlockSpec((D, 2 * D), lambda i: (0, 0)),
            pl.BlockSpec((1, 2 * D), lambda i: (0, 0)),
            pl.BlockSpec((D, 2 * D), lambda i: (0, 0)),
        ],
        out_specs=[
            pl.BlockSpec((BN, 2 * D), lambda i: (i, 0)),
            pl.BlockSpec((BN, 2 * D), lambda i: (i, 0)),
        ],
        out_shape=[
            jax.ShapeDtypeStruct((N, 2 * D), jnp.bfloat16),
            jax.ShapeDtypeStruct((N, 2 * D), jnp.bfloat16),
        ],
    )(nf, s_attn, wcat_t, bcat_t, wcat_s)


def _ve_call(ef, wve, bv):
    grid = E // BE
    return pl.pallas_call(
        _ve_body,
        grid=(grid,),
        in_specs=[
            pl.BlockSpec((BE, DE), lambda i: (i, 0)),
            pl.BlockSpec((DE, D), lambda i: (0, 0)),
            pl.BlockSpec((1, D), lambda i: (0, 0)),
        ],
        out_specs=pl.BlockSpec((BE, D), lambda i: (i, 0)),
        out_shape=jax.ShapeDtypeStruct((E, D), jnp.bfloat16),
    )(ef, wve, bv)


def _post_call(acc, nf, wo, bo, srep, s_ffn, w1, w2):
    grid = N // BN
    return pl.pallas_call(
        _post_body,
        grid=(grid,),
        in_specs=[
            pl.BlockSpec((2, BN, PAY), lambda i: (0, i, 0)),
            pl.BlockSpec((BN, D), lambda i: (i, 0)),
            pl.BlockSpec((D, D), lambda i: (0, 0)),
            pl.BlockSpec((1, D), lambda i: (0, 0)),
            pl.BlockSpec((PAY, D), lambda i: (0, 0)),
            pl.BlockSpec((1, D), lambda i: (0, 0)),
            pl.BlockSpec((D, FFN), lambda i: (0, 0)),
            pl.BlockSpec((FFN, D), lambda i: (0, 0)),
        ],
        out_specs=pl.BlockSpec((BN, D), lambda i: (i, 0)),
        out_shape=jax.ShapeDtypeStruct((N, D), jnp.float32),
    )(acc, nf, wo, bo, srep, s_ffn, w1, w2)


_sc_edge_call = functools.partial(
    pl.kernel,
    out_type=jax.ShapeDtypeStruct((NC, N, PAY), jnp.float32),
    mesh=plsc.VectorSubcoreMesh(core_axis_name="c", subcore_axis_name="s"),
    compiler_params=pltpu.CompilerParams(use_tc_tiling_on_sc=False,
                                         needs_layout_passes=False),
    scratch_types=(
        [pltpu.VMEM((CH,), jnp.int32)] * 4
        + [pltpu.VMEM((CH, D // 2), jnp.int32)] * 2
        + [pltpu.VMEM((CH, DE), jnp.float32)] * 2
        + [pltpu.VMEM((CH, D), jnp.int32)] * 4
        + [pltpu.VMEM((CH, PAY), jnp.float32)]
        + [pltpu.VMEM_SHARED((N, PAY), jnp.float32)]
        + [pltpu.SemaphoreType.DMA] * 12
    ),
)(_sc_edge_body)


def kernel(node_feats, edge_feats, edge_index, Wq, bq, Wk, bk, Wv, bv,
           Wo, bo, s_attn, s_ffn, W1, W2):
    src = edge_index[0]
    tgt = edge_index[1]
    we = Wk[D:].reshape(DE, H, C)
    wblk = jnp.einsum('jhc,hg->hcgj', we, jnp.eye(H, dtype=jnp.float32))
    wblk = wblk.reshape(H * C, H * DE)
    wq_s = Wq * INV_SQRT_C
    bq_s = bq * INV_SQRT_C
    wcat_t = jnp.concatenate([wq_s, wq_s @ wblk], axis=1)[:, _PERM_T]
    bcat_t = jnp.concatenate([bq_s, bq_s @ wblk])[_PERM_T].reshape(1, 2 * D)
    wcat_s = jnp.concatenate([Wk[:D], Wv[:D]], axis=1)[:, _PERM_T]
    wve = Wv[D:][:, _PERM_V]
    bv_p = bv[_PERM_V].reshape(1, D)
    srep = jnp.concatenate(
        [jnp.zeros((D, D), jnp.float32),
         jnp.kron(jnp.eye(H, dtype=jnp.float32), jnp.ones((1, C), jnp.float32)),
         jnp.zeros((PAY - D - H, D), jnp.float32)], axis=0)

    t_tgt, t_src = _pre_call(node_feats, s_attn.reshape(1, D),
                             wcat_t, bcat_t, wcat_s)
    ve = _ve_call(edge_feats, wve, bv_p)
    t_tgt_i = lax.bitcast_convert_type(t_tgt.reshape(N, D, 2), jnp.int32)
    t_src_i = lax.bitcast_convert_type(t_src.reshape(N, D, 2), jnp.int32)
    ve_i = lax.bitcast_convert_type(ve.reshape(E, D // 2, 2), jnp.int32)
    acc = _sc_edge_call(t_tgt_i, t_src_i, ve_i, edge_feats, src, tgt)
    out = _post_call(acc, node_feats, Wo, bo.reshape(1, D), srep,
                     s_ffn.reshape(1, D), W1, W2)
    return out


# speedup vs baseline: 2.7503x; 1.8795x over previous
import functools
import math

import jax
import jax.numpy as jnp
import numpy as np
from jax import lax
from jax.experimental import pallas as pl
from jax.experimental.pallas import tpu as pltpu
from jax.experimental.pallas import tpu_sc as plsc

N = 10000
E = 320000
D = 128
DE = 16
H = 8
C = 16
FFN = 512
EPS = 1e-8

PAY = 144
NC, NS = 2, 16
NW = NC * NS
EPT = E // NW
CH = 40
NG = EPT // CH
ROWS_PT = N // NS
SQRT_D = math.sqrt(D)
INV_SQRT_C = 1.0 / math.sqrt(C)

BN = 400
BE = 3200

_ILV = plsc.PackFormat.INTERLEAVED

_PERM_T = np.empty(2 * D, np.int32)
for _h in range(H):
    for _k in range(C):
        _PERM_T[32 * _h + 2 * _k] = 16 * _h + _k
        _PERM_T[32 * _h + 2 * _k + 1] = D + 16 * _h + _k
_PERM_V = np.empty(D, np.int32)
for _j in range(H // 2):
    for _k in range(C):
        _PERM_V[32 * _j + 2 * _k] = 32 * _j + _k
        _PERM_V[32 * _j + 2 * _k + 1] = 32 * _j + 16 + _k


def _bcast_lane(v, h):
    idx = jnp.full((16,), h, jnp.int32)
    return v.at[idx].get(mode="promise_in_bounds")


def _pre_body(nf, s_attn, wcat_t, bcat_t, wcat_s, t_tgt, t_src):
    x = nf[...]
    nrm = jnp.sqrt(jnp.sum(x * x, axis=1, keepdims=True))
    h = s_attn[...] * x / (nrm / SQRT_D + EPS)
    t_tgt[...] = (jnp.dot(h, wcat_t[...], preferred_element_type=jnp.float32)
                  + bcat_t[...]).astype(jnp.bfloat16)
    t_src[...] = jnp.dot(h, wcat_s[...],
                         preferred_element_type=jnp.float32).astype(jnp.bfloat16)


def _ve_body(ef, wve, bv, ve):
    ve[...] = (jnp.dot(ef[...], wve[...], preferred_element_type=jnp.float32)
               + bv[...]).astype(jnp.bfloat16)


def _post_body(acc, nf, wo, bo, srep, s_ffn, w1, w2, out):
    a = acc[0] + acc[1]
    arep = jnp.dot(a, srep[...], preferred_element_type=jnp.float32)
    attn = a[:, :D] * (1.0 / (arep + 1e-16))
    y = jnp.dot(attn, wo[...], preferred_element_type=jnp.float32) + bo[...]
    x1 = nf[...] + y
    nrm = jnp.sqrt(jnp.sum(x1 * x1, axis=1, keepdims=True))
    h2 = s_ffn[...] * x1 / (nrm / SQRT_D + EPS)
    g = jax.nn.gelu(jnp.dot(h2, w1[...], preferred_element_type=jnp.float32))
    out[...] = x1 + jnp.dot(g, w2[...], preferred_element_type=jnp.float32)


def _sc_edge_body(t_tgt, t_src, ve_hbm, ef_hbm, src_hbm, tgt_hbm, out_hbm,
                  sidx0, sidx1, tidx0, tidx1, ve0, ve1, ef0, ef1,
                  rt0, rt1, rs0, rs1, pay, acc,
                  s_si0, s_si1, s_ti0, s_ti1, s_ve0, s_ve1, s_ef0, s_ef1,
                  s_rt0, s_rt1, s_rs0, s_rs1):
    c = lax.axis_index("c")
    s = lax.axis_index("s")
    wid = c * NS + s
    zero16 = jnp.zeros((16,), jnp.float32)
    lane = lax.iota(jnp.int32, 16)
    ivef = ((sidx0, tidx0, ve0, ef0, s_si0, s_ti0, s_ve0, s_ef0),
            (sidx1, tidx1, ve1, ef1, s_si1, s_ti1, s_ve1, s_ef1))
    rows = ((rt0, rs0, s_rt0, s_rs0), (rt1, rs1, s_rt1, s_rs1))

    def zrow(i, carry):
        for j in range(PAY // 16):
            pay[i, pl.ds(j * 16, 16)] = zero16
        return carry

    lax.fori_loop(0, CH, zrow, None)
    rowbase = s * ROWS_PT
    nfull = ROWS_PT // CH
    rem = ROWS_PT - nfull * CH
    for j in range(nfull):
        pltpu.sync_copy(pay.at[pl.ds(0, CH)],
                        acc.at[pl.ds(rowbase + j * CH, CH)])
    if rem:
        pltpu.sync_copy(pay.at[pl.ds(0, rem)],
                        acc.at[pl.ds(rowbase + nfull * CH, rem)])
    plsc.subcore_barrier()

    ebase = wid * EPT

    def issue_ivef(off, b):
        si, ti, ve, ef, ssi, sti, sve, sef = ivef[b]
        pltpu.async_copy(src_hbm.at[pl.ds(off, CH)], si, ssi)
        pltpu.async_copy(tgt_hbm.at[pl.ds(off, CH)], ti, sti)
        pltpu.async_copy(ve_hbm.at[pl.ds(off, CH)], ve, sve)
        pltpu.async_copy(ef_hbm.at[pl.ds(off, CH)], ef, sef)

    def wait_idx(off, b):
        si, ti, _, _, ssi, sti, _, _ = ivef[b]
        pltpu.make_async_copy(src_hbm.at[pl.ds(off, CH)], si, ssi).wait()
        pltpu.make_async_copy(tgt_hbm.at[pl.ds(off, CH)], ti, sti).wait()

    def issue_rows(b):
        si, ti = ivef[b][0], ivef[b][1]
        rt, rs, srt, srs = rows[b]
        pltpu.async_copy(t_tgt.at[ti], rt, srt)
        pltpu.async_copy(t_src.at[si], rs, srs)

    def wait_rows(b):
        si, ti = ivef[b][0], ivef[b][1]
        rt, rs, srt, srs = rows[b]
        pltpu.make_async_copy(t_tgt.at[ti], rt, srt).wait()
        pltpu.make_async_copy(t_src.at[si], rs, srs).wait()

    def compute(b, off):
        _, _, ve_v, ef_v, _, _, sve, sef = ivef[b]
        rt, rs, _, _ = rows[b]
        pltpu.make_async_copy(ve_hbm.at[pl.ds(off, CH)], ve_v, sve).wait()
        pltpu.make_async_copy(ef_hbm.at[pl.ds(off, CH)], ef_v, sef).wait()
        wait_rows(b)

        def edge2(k):
            for u in range(4):
                i = k + u
                efe = ef_v[i, :]
                pc = zero16
                vns = []
                for h in range(H):
                    qt, bt = plsc.unpack(rt[i, pl.ds(32 * h, 32)], format=_ILV,
                                         preferred_element_type=jnp.float32)
                    ks, vn = plsc.unpack(rs[i, pl.ds(32 * h, 32)], format=_ILV,
                                         preferred_element_type=jnp.float32)
                    vns.append(vn)
                    lg = jnp.sum(qt * ks + bt * efe)
                    pc = jnp.where(lane == h, lg, pc)
                pv = jnp.exp(pc)
                pay[i, pl.ds(128, 16)] = pv
                for j in range(H // 2):
                    va, vb = plsc.unpack(ve_v[i, pl.ds(32 * j, 32)], format=_ILV,
                                         preferred_element_type=jnp.float32)
                    for p, veh in ((0, va), (1, vb)):
                        h = 2 * j + p
                        pb = _bcast_lane(pv, h)
                        pay[i, pl.ds(16 * h, 16)] = pb * (vns[h] + veh)

        plsc.parallel_loop(0, CH, 4)(edge2)

    def body(g, b, pre1, pre2):
        off = ebase + g * CH
        if pre1:
            wait_idx(off + CH, 1 - b)
            issue_rows(1 - b)
        compute(b, off)
        pltpu.sync_copy(pay, acc.at[ivef[b][1]], add=True)
        if pre2:
            issue_ivef(off + 2 * CH, b)

    issue_ivef(ebase, 0)
    wait_idx(ebase, 0)
    issue_rows(0)
    issue_ivef(ebase + CH, 1)

    def pair(gp, carry):
        body(2 * gp, 0, True, True)
        body(2 * gp + 1, 1, True, True)
        return carry

    lax.fori_loop(0, NG // 2 - 1, pair, None)
    body(NG - 2, 0, True, False)
    body(NG - 1, 1, False, False)
    plsc.subcore_barrier()
    for j in range(nfull):
        pltpu.sync_copy(acc.at[pl.ds(rowbase + j * CH, CH)],
                        out_hbm.at[c, pl.ds(rowbase + j * CH, CH)])
    if rem:
        pltpu.sync_copy(acc.at[pl.ds(rowbase + nfull * CH, rem)],
                        out_hbm.at[c, pl.ds(rowbase + nfull * CH, rem)])


def _pre_call(nf, s_attn, wcat_t, bcat_t, wcat_s):
    grid = N // BN
    return pl.pallas_call(
        _pre_body,
        grid=(grid,),
        in_specs=[
            pl.BlockSpec((BN, D), lambda i: (i, 0)),
            pl.BlockSpec((1, D), lambda i: (0, 0)),
            pl.BlockSpec((D, 2 * D), lambda i: (0, 0)),
            pl.BlockSpec((1, 2 * D), lambda i: (0, 0)),
            pl.BlockSpec((D, 2 * D), lambda i: (0, 0)),
        ],
        out_specs=[
            pl.BlockSpec((BN, 2 * D), lambda i: (i, 0)),
            pl.BlockSpec((BN, 2 * D), lambda i: (i, 0)),
        ],
        out_shape=[
            jax.ShapeDtypeStruct((N, 2 * D), jnp.bfloat16),
            jax.ShapeDtypeStruct((N, 2 * D), jnp.bfloat16),
        ],
    )(nf, s_attn, wcat_t, bcat_t, wcat_s)


def _ve_call(ef, wve, bv):
    grid = E // BE
    return pl.pallas_call(
        _ve_body,
        grid=(grid,),
        in_specs=[
            pl.BlockSpec((BE, DE), lambda i: (i, 0)),
            pl.BlockSpec((DE, D), lambda i: (0, 0)),
            pl.BlockSpec((1, D), lambda i: (0, 0)),
        ],
        out_specs=pl.BlockSpec((BE, D), lambda i: (i, 0)),
        out_shape=jax.ShapeDtypeStruct((E, D), jnp.bfloat16),
    )(ef, wve, bv)


def _post_call(acc, nf, wo, bo, srep, s_ffn, w1, w2):
    grid = N // BN
    return pl.pallas_call(
        _post_body,
        grid=(grid,),
        in_specs=[
            pl.BlockSpec((2, BN, PAY), lambda i: (0, i, 0)),
            pl.BlockSpec((BN, D), lambda i: (i, 0)),
            pl.BlockSpec((D, D), lambda i: (0, 0)),
            pl.BlockSpec((1, D), lambda i: (0, 0)),
            pl.BlockSpec((PAY, D), lambda i: (0, 0)),
            pl.BlockSpec((1, D), lambda i: (0, 0)),
            pl.BlockSpec((D, FFN), lambda i: (0, 0)),
            pl.BlockSpec((FFN, D), lambda i: (0, 0)),
        ],
        out_specs=pl.BlockSpec((BN, D), lambda i: (i, 0)),
        out_shape=jax.ShapeDtypeStruct((N, D), jnp.float32),
    )(acc, nf, wo, bo, srep, s_ffn, w1, w2)


_sc_edge_call = functools.partial(
    pl.kernel,
    out_type=jax.ShapeDtypeStruct((NC, N, PAY), jnp.float32),
    mesh=plsc.VectorSubcoreMesh(core_axis_name="c", subcore_axis_name="s"),
    compiler_params=pltpu.CompilerParams(use_tc_tiling_on_sc=False,
                                         needs_layout_passes=False),
    scratch_types=(
        [pltpu.VMEM((CH,), jnp.int32)] * 4
        + [pltpu.VMEM((CH, D), jnp.bfloat16)] * 2
        + [pltpu.VMEM((CH, DE), jnp.float32)] * 2
        + [pltpu.VMEM((CH, 2 * D), jnp.bfloat16)] * 4
        + [pltpu.VMEM((CH, PAY), jnp.float32)]
        + [pltpu.VMEM_SHARED((N, PAY), jnp.float32)]
        + [pltpu.SemaphoreType.DMA] * 12
    ),
)(_sc_edge_body)


def kernel(node_feats, edge_feats, edge_index, Wq, bq, Wk, bk, Wv, bv,
           Wo, bo, s_attn, s_ffn, W1, W2):
    src = edge_index[0]
    tgt = edge_index[1]
    we = Wk[D:].reshape(DE, H, C)
    wblk = jnp.einsum('jhc,hg->hcgj', we, jnp.eye(H, dtype=jnp.float32))
    wblk = wblk.reshape(H * C, H * DE)
    wq_s = Wq * INV_SQRT_C
    bq_s = bq * INV_SQRT_C
    wcat_t = jnp.concatenate([wq_s, wq_s @ wblk], axis=1)[:, _PERM_T]
    bcat_t = jnp.concatenate([bq_s, bq_s @ wblk])[_PERM_T].reshape(1, 2 * D)
    wcat_s = jnp.concatenate([Wk[:D], Wv[:D]], axis=1)[:, _PERM_T]
    wve = Wv[D:][:, _PERM_V]
    bv_p = bv[_PERM_V].reshape(1, D)
    srep = jnp.concatenate(
        [jnp.zeros((D, D), jnp.float32),
         jnp.kron(jnp.eye(H, dtype=jnp.float32), jnp.ones((1, C), jnp.float32)),
         jnp.zeros((PAY - D - H, D), jnp.float32)], axis=0)

    t_tgt, t_src = _pre_call(node_feats, s_attn.reshape(1, D),
                             wcat_t, bcat_t, wcat_s)
    ve = _ve_call(edge_feats, wve, bv_p)
    acc = _sc_edge_call(t_tgt, t_src, ve, edge_feats, src, tgt)
    out = _post_call(acc, node_feats, Wo, bo.reshape(1, D), srep,
                     s_ffn.reshape(1, D), W1, W2)
    return out


# speedup vs baseline: 3.0966x; 1.1259x over previous
import functools
import math

import jax
import jax.numpy as jnp
import numpy as np
from jax import lax
from jax.experimental import pallas as pl
from jax.experimental.pallas import tpu as pltpu
from jax.experimental.pallas import tpu_sc as plsc

N = 10000
E = 320000
D = 128
DE = 16
H = 8
C = 16
FFN = 512
EPS = 1e-8

PAY = 144
NC, NS = 2, 16
NW = NC * NS
EPT = E // NW
CH = 40
NG = EPT // CH
ROWS_PT = N // NS
SQRT_D = math.sqrt(D)
INV_SQRT_C = 1.0 / math.sqrt(C)

BN = 400
BE = 3200

_ILV = plsc.PackFormat.INTERLEAVED

_PERM_T = np.empty(2 * D, np.int32)
for _h in range(H):
    for _k in range(C):
        _PERM_T[32 * _h + 2 * _k] = 16 * _h + _k
        _PERM_T[32 * _h + 2 * _k + 1] = D + 16 * _h + _k
_PERM_V = np.empty(D, np.int32)
for _j in range(H // 2):
    for _k in range(C):
        _PERM_V[32 * _j + 2 * _k] = 32 * _j + _k
        _PERM_V[32 * _j + 2 * _k + 1] = 32 * _j + 16 + _k


def _bcast_lane(v, h):
    idx = jnp.full((16,), h, jnp.int32)
    return v.at[idx].get(mode="promise_in_bounds")


def _pre_body(nf, s_attn, wcat_t, bcat_t, wcat_s, t_tgt, t_src):
    x = nf[...]
    nrm = jnp.sqrt(jnp.sum(x * x, axis=1, keepdims=True))
    h = s_attn[...] * x / (nrm / SQRT_D + EPS)
    t_tgt[...] = (jnp.dot(h, wcat_t[...], preferred_element_type=jnp.float32)
                  + bcat_t[...]).astype(jnp.bfloat16)
    t_src[...] = jnp.dot(h, wcat_s[...],
                         preferred_element_type=jnp.float32).astype(jnp.bfloat16)


def _ve_body(ef, wve, bv, ve):
    ve[...] = (jnp.dot(ef[...], wve[...], preferred_element_type=jnp.float32)
               + bv[...]).astype(jnp.bfloat16)


def _post_body(acc, nf, wo, bo, srep, s_ffn, w1, w2, out):
    a = acc[0] + acc[1]
    arep = jnp.dot(a, srep[...], preferred_element_type=jnp.float32)
    attn = a[:, :D] * (1.0 / (arep + 1e-16))
    y = jnp.dot(attn, wo[...], preferred_element_type=jnp.float32) + bo[...]
    x1 = nf[...] + y
    nrm = jnp.sqrt(jnp.sum(x1 * x1, axis=1, keepdims=True))
    h2 = s_ffn[...] * x1 / (nrm / SQRT_D + EPS)
    g = jax.nn.gelu(jnp.dot(h2, w1[...], preferred_element_type=jnp.float32))
    out[...] = x1 + jnp.dot(g, w2[...], preferred_element_type=jnp.float32)


def _sc_edge_body(t_tgt, t_src, ve_hbm, ef_hbm, src_hbm, tgt_hbm, out_hbm,
                  sidx0, sidx1, tidx0, tidx1, ve0, ve1, ef0, ef1,
                  rt0, rt1, rs0, rs1, pay, acc,
                  s_si0, s_si1, s_ti0, s_ti1, s_ve0, s_ve1, s_ef0, s_ef1,
                  s_rt0, s_rt1, s_rs0, s_rs1):
    c = lax.axis_index("c")
    s = lax.axis_index("s")
    wid = c * NS + s
    zero16 = jnp.zeros((16,), jnp.float32)
    lane = lax.iota(jnp.int32, 16)
    ivef = ((sidx0, tidx0, ve0, ef0, s_si0, s_ti0, s_ve0, s_ef0),
            (sidx1, tidx1, ve1, ef1, s_si1, s_ti1, s_ve1, s_ef1))
    rows = ((rt0, rs0, s_rt0, s_rs0), (rt1, rs1, s_rt1, s_rs1))

    def zrow(i, carry):
        for j in range(PAY // 16):
            pay[i, pl.ds(j * 16, 16)] = zero16
        return carry

    lax.fori_loop(0, CH, zrow, None)
    rowbase = s * ROWS_PT
    nfull = ROWS_PT // CH
    rem = ROWS_PT - nfull * CH
    for j in range(nfull):
        pltpu.sync_copy(pay.at[pl.ds(0, CH)],
                        acc.at[pl.ds(rowbase + j * CH, CH)])
    if rem:
        pltpu.sync_copy(pay.at[pl.ds(0, rem)],
                        acc.at[pl.ds(rowbase + nfull * CH, rem)])
    plsc.subcore_barrier()

    ebase = wid * EPT

    def issue_ivef(off, b):
        si, ti, ve, ef, ssi, sti, sve, sef = ivef[b]
        pltpu.async_copy(src_hbm.at[pl.ds(off, CH)], si, ssi)
        pltpu.async_copy(tgt_hbm.at[pl.ds(off, CH)], ti, sti)
        pltpu.async_copy(ve_hbm.at[pl.ds(off, CH)], ve, sve)
        pltpu.async_copy(ef_hbm.at[pl.ds(off, CH)], ef, sef)

    def wait_idx(off, b):
        si, ti, _, _, ssi, sti, _, _ = ivef[b]
        pltpu.make_async_copy(src_hbm.at[pl.ds(off, CH)], si, ssi).wait()
        pltpu.make_async_copy(tgt_hbm.at[pl.ds(off, CH)], ti, sti).wait()

    def issue_rows(b):
        si, ti = ivef[b][0], ivef[b][1]
        rt, rs, srt, srs = rows[b]
        pltpu.async_copy(t_tgt.at[ti], rt, srt)
        pltpu.async_copy(t_src.at[si], rs, srs)

    def wait_rows(b):
        si, ti = ivef[b][0], ivef[b][1]
        rt, rs, srt, srs = rows[b]
        pltpu.make_async_copy(t_tgt.at[ti], rt, srt).wait()
        pltpu.make_async_copy(t_src.at[si], rs, srs).wait()

    def compute(b, off):
        _, _, ve_v, ef_v, _, _, sve, sef = ivef[b]
        rt, rs, _, _ = rows[b]
        pltpu.make_async_copy(ve_hbm.at[pl.ds(off, CH)], ve_v, sve).wait()
        pltpu.make_async_copy(ef_hbm.at[pl.ds(off, CH)], ef_v, sef).wait()
        wait_rows(b)

        def edge2(k):
            for u in range(2):
                i = k + u
                efe = ef_v[i, :]
                pc = zero16
                vns = []
                for h in range(H):
                    qt, bt = plsc.unpack(rt[i, pl.ds(32 * h, 32)], format=_ILV,
                                         preferred_element_type=jnp.float32)
                    ks, vn = plsc.unpack(rs[i, pl.ds(32 * h, 32)], format=_ILV,
                                         preferred_element_type=jnp.float32)
                    vns.append(vn)
                    lg = jnp.sum(qt * ks + bt * efe)
                    pc = jnp.where(lane == h, lg, pc)
                pv = jnp.exp(pc)
                pay[i, pl.ds(128, 16)] = pv
                for j in range(H // 2):
                    va, vb = plsc.unpack(ve_v[i, pl.ds(32 * j, 32)], format=_ILV,
                                         preferred_element_type=jnp.float32)
                    for p, veh in ((0, va), (1, vb)):
                        h = 2 * j + p
                        pb = _bcast_lane(pv, h)
                        pay[i, pl.ds(16 * h, 16)] = pb * (vns[h] + veh)

        plsc.parallel_loop(0, CH, 2)(edge2)

    def body(g, b, pre1, pre2):
        off = ebase + g * CH
        if pre1:
            wait_idx(off + CH, 1 - b)
            issue_rows(1 - b)
        compute(b, off)
        pltpu.sync_copy(pay, acc.at[ivef[b][1]], add=True)
        if pre2:
            issue_ivef(off + 2 * CH, b)

    issue_ivef(ebase, 0)
    wait_idx(ebase, 0)
    issue_rows(0)
    issue_ivef(ebase + CH, 1)

    def pair(gp, carry):
        body(2 * gp, 0, True, True)
        body(2 * gp + 1, 1, True, True)
        return carry

    lax.fori_loop(0, NG // 2 - 1, pair, None)
    body(NG - 2, 0, True, False)
    body(NG - 1, 1, False, False)
    plsc.subcore_barrier()
    for j in range(nfull):
        pltpu.sync_copy(acc.at[pl.ds(rowbase + j * CH, CH)],
                        out_hbm.at[c, pl.ds(rowbase + j * CH, CH)])
    if rem:
        pltpu.sync_copy(acc.at[pl.ds(rowbase + nfull * CH, rem)],
                        out_hbm.at[c, pl.ds(rowbase + nfull * CH, rem)])


def _pre_call(nf, s_attn, wcat_t, bcat_t, wcat_s):
    grid = N // BN
    return pl.pallas_call(
        _pre_body,
        grid=(grid,),
        in_specs=[
            pl.BlockSpec((BN, D), lambda i: (i, 0)),
            pl.BlockSpec((1, D), lambda i: (0, 0)),
            pl.BlockSpec((D, 2 * D), lambda i: (0, 0)),
            pl.BlockSpec((1, 2 * D), lambda i: (0, 0)),
            pl.BlockSpec((D, 2 * D), lambda i: (0, 0)),
        ],
        out_specs=[
            pl.BlockSpec((BN, 2 * D), lambda i: (i, 0)),
            pl.BlockSpec((BN, 2 * D), lambda i: (i, 0)),
        ],
        out_shape=[
            jax.ShapeDtypeStruct((N, 2 * D), jnp.bfloat16),
            jax.ShapeDtypeStruct((N, 2 * D), jnp.bfloat16),
        ],
    )(nf, s_attn, wcat_t, bcat_t, wcat_s)


def _ve_call(ef, wve, bv):
    grid = E // BE
    return pl.pallas_call(
        _ve_body,
        grid=(grid,),
        in_specs=[
            pl.BlockSpec((BE, DE), lambda i: (i, 0)),
            pl.BlockSpec((DE, D), lambda i: (0, 0)),
            pl.BlockSpec((1, D), lambda i: (0, 0)),
        ],
        out_specs=pl.BlockSpec((BE, D), lambda i: (i, 0)),
        out_shape=jax.ShapeDtypeStruct((E, D), jnp.bfloat16),
    )(ef, wve, bv)


def _post_call(acc, nf, wo, bo, srep, s_ffn, w1, w2):
    grid = N // BN
    return pl.pallas_call(
        _post_body,
        grid=(grid,),
        in_specs=[
            pl.BlockSpec((2, BN, PAY), lambda i: (0, i, 0)),
            pl.BlockSpec((BN, D), lambda i: (i, 0)),
            pl.BlockSpec((D, D), lambda i: (0, 0)),
            pl.BlockSpec((1, D), lambda i: (0, 0)),
            pl.BlockSpec((PAY, D), lambda i: (0, 0)),
            pl.BlockSpec((1, D), lambda i: (0, 0)),
            pl.BlockSpec((D, FFN), lambda i: (0, 0)),
            pl.BlockSpec((FFN, D), lambda i: (0, 0)),
        ],
        out_specs=pl.BlockSpec((BN, D), lambda i: (i, 0)),
        out_shape=jax.ShapeDtypeStruct((N, D), jnp.float32),
    )(acc, nf, wo, bo, srep, s_ffn, w1, w2)


_sc_edge_call = functools.partial(
    pl.kernel,
    out_type=jax.ShapeDtypeStruct((NC, N, PAY), jnp.float32),
    mesh=plsc.VectorSubcoreMesh(core_axis_name="c", subcore_axis_name="s"),
    compiler_params=pltpu.CompilerParams(use_tc_tiling_on_sc=False,
                                         needs_layout_passes=False),
    scratch_types=(
        [pltpu.VMEM((CH,), jnp.int32)] * 4
        + [pltpu.VMEM((CH, D), jnp.bfloat16)] * 2
        + [pltpu.VMEM((CH, DE), jnp.float32)] * 2
        + [pltpu.VMEM((CH, 2 * D), jnp.bfloat16)] * 4
        + [pltpu.VMEM((CH, PAY), jnp.float32)]
        + [pltpu.VMEM_SHARED((N, PAY), jnp.float32)]
        + [pltpu.SemaphoreType.DMA] * 12
    ),
)(_sc_edge_body)


def kernel(node_feats, edge_feats, edge_index, Wq, bq, Wk, bk, Wv, bv,
           Wo, bo, s_attn, s_ffn, W1, W2):
    src = edge_index[0]
    tgt = edge_index[1]
    we = Wk[D:].reshape(DE, H, C)
    wblk = jnp.einsum('jhc,hg->hcgj', we, jnp.eye(H, dtype=jnp.float32))
    wblk = wblk.reshape(H * C, H * DE)
    wq_s = Wq * INV_SQRT_C
    bq_s = bq * INV_SQRT_C
    wcat_t = jnp.concatenate([wq_s, wq_s @ wblk], axis=1)[:, _PERM_T]
    bcat_t = jnp.concatenate([bq_s, bq_s @ wblk])[_PERM_T].reshape(1, 2 * D)
    wcat_s = jnp.concatenate([Wk[:D], Wv[:D]], axis=1)[:, _PERM_T]
    wve = Wv[D:][:, _PERM_V]
    bv_p = bv[_PERM_V].reshape(1, D)
    srep = jnp.concatenate(
        [jnp.zeros((D, D), jnp.float32),
         jnp.kron(jnp.eye(H, dtype=jnp.float32), jnp.ones((1, C), jnp.float32)),
         jnp.zeros((PAY - D - H, D), jnp.float32)], axis=0)

    t_tgt, t_src = _pre_call(node_feats, s_attn.reshape(1, D),
                             wcat_t, bcat_t, wcat_s)
    ve = _ve_call(edge_feats, wve, bv_p)
    acc = _sc_edge_call(t_tgt, t_src, ve, edge_feats, src, tgt)
    out = _post_call(acc, node_feats, Wo, bo.reshape(1, D), srep,
                     s_ffn.reshape(1, D), W1, W2)
    return out


# speedup vs baseline: 3.1419x; 1.0146x over previous
import functools
import math

import jax
import jax.numpy as jnp
import numpy as np
from jax import lax
from jax.experimental import pallas as pl
from jax.experimental.pallas import tpu as pltpu
from jax.experimental.pallas import tpu_sc as plsc

N = 10000
E = 320000
D = 128
DE = 16
H = 8
C = 16
FFN = 512
EPS = 1e-8

PAY = 144
NC, NS = 2, 16
NW = NC * NS
EPT = E // NW
CH = 40
NG = EPT // CH
ROWS_PT = N // NS
SQRT_D = math.sqrt(D)
INV_SQRT_C = 1.0 / math.sqrt(C)

BN = 400
BE = 3200

_ILV = plsc.PackFormat.INTERLEAVED

_PERM_T = np.empty(2 * D, np.int32)
for _h in range(H):
    for _k in range(C):
        _PERM_T[32 * _h + 2 * _k] = 16 * _h + _k
        _PERM_T[32 * _h + 2 * _k + 1] = D + 16 * _h + _k
_PERM_V = np.empty(D, np.int32)
for _j in range(H // 2):
    for _k in range(C):
        _PERM_V[32 * _j + 2 * _k] = 32 * _j + _k
        _PERM_V[32 * _j + 2 * _k + 1] = 32 * _j + 16 + _k


def _bcast_lane(v, h):
    idx = jnp.full((16,), h, jnp.int32)
    return v.at[idx].get(mode="promise_in_bounds")


def _pre_body(nf, s_attn, wcat_t, bcat_t, wcat_s, t_tgt, t_src):
    x = nf[...]
    nrm = jnp.sqrt(jnp.sum(x * x, axis=1, keepdims=True))
    h = s_attn[...] * x / (nrm / SQRT_D + EPS)
    t_tgt[...] = (jnp.dot(h, wcat_t[...], preferred_element_type=jnp.float32)
                  + bcat_t[...]).astype(jnp.bfloat16)
    t_src[...] = jnp.dot(h, wcat_s[...],
                         preferred_element_type=jnp.float32).astype(jnp.bfloat16)


def _ve_body(ef, wve, bv, ve):
    ve[...] = (jnp.dot(ef[...], wve[...], preferred_element_type=jnp.float32)
               + bv[...]).astype(jnp.bfloat16)


def _post_body(acc, nf, wo, bo, srep, s_ffn, w1, w2, out):
    a = acc[0] + acc[1]
    arep = jnp.dot(a, srep[...], preferred_element_type=jnp.float32)
    attn = a[:, :D] * (1.0 / (arep + 1e-16))
    y = jnp.dot(attn, wo[...], preferred_element_type=jnp.float32) + bo[...]
    x1 = nf[...] + y
    nrm = jnp.sqrt(jnp.sum(x1 * x1, axis=1, keepdims=True))
    h2 = s_ffn[...] * x1 / (nrm / SQRT_D + EPS)
    g = jax.nn.gelu(jnp.dot(h2, w1[...], preferred_element_type=jnp.float32))
    out[...] = x1 + jnp.dot(g, w2[...], preferred_element_type=jnp.float32)


def _sc_edge_body(t_tgt, t_src, ve_hbm, ef_hbm, src_hbm, tgt_hbm, out_hbm,
                  sidx0, sidx1, tidx0, tidx1, ve0, ve1, ef0, ef1,
                  rt0, rt1, rs0, rs1, pay, acc,
                  s_si0, s_si1, s_ti0, s_ti1, s_ve0, s_ve1, s_ef0, s_ef1,
                  s_rt0, s_rt1, s_rs0, s_rs1):
    c = lax.axis_index("c")
    s = lax.axis_index("s")
    wid = c * NS + s
    zero16 = jnp.zeros((16,), jnp.float32)
    lane = lax.iota(jnp.int32, 16)
    ivef = ((sidx0, tidx0, ve0, ef0, s_si0, s_ti0, s_ve0, s_ef0),
            (sidx1, tidx1, ve1, ef1, s_si1, s_ti1, s_ve1, s_ef1))
    rows = ((rt0, rs0, s_rt0, s_rs0), (rt1, rs1, s_rt1, s_rs1))

    def zrow(i, carry):
        for j in range(PAY // 16):
            pay[i, pl.ds(j * 16, 16)] = zero16
        return carry

    lax.fori_loop(0, CH, zrow, None)
    rowbase = s * ROWS_PT
    nfull = ROWS_PT // CH
    rem = ROWS_PT - nfull * CH
    for j in range(nfull):
        pltpu.sync_copy(pay.at[pl.ds(0, CH)],
                        acc.at[pl.ds(rowbase + j * CH, CH)])
    if rem:
        pltpu.sync_copy(pay.at[pl.ds(0, rem)],
                        acc.at[pl.ds(rowbase + nfull * CH, rem)])
    plsc.subcore_barrier()

    ebase = wid * EPT

    def issue_ivef(off, b):
        si, ti, ve, ef, ssi, sti, sve, sef = ivef[b]
        pltpu.async_copy(src_hbm.at[pl.ds(off, CH)], si, ssi)
        pltpu.async_copy(tgt_hbm.at[pl.ds(off, CH)], ti, sti)
        pltpu.async_copy(ve_hbm.at[pl.ds(off, CH)], ve, sve)
        pltpu.async_copy(ef_hbm.at[pl.ds(off, CH)], ef, sef)

    def wait_idx(off, b):
        si, ti, _, _, ssi, sti, _, _ = ivef[b]
        pltpu.make_async_copy(src_hbm.at[pl.ds(off, CH)], si, ssi).wait()
        pltpu.make_async_copy(tgt_hbm.at[pl.ds(off, CH)], ti, sti).wait()

    def issue_rows(b):
        si, ti = ivef[b][0], ivef[b][1]
        rt, rs, srt, srs = rows[b]
        pltpu.async_copy(t_tgt.at[ti], rt, srt)
        pltpu.async_copy(t_src.at[si], rs, srs)

    def wait_rows(b):
        si, ti = ivef[b][0], ivef[b][1]
        rt, rs, srt, srs = rows[b]
        pltpu.make_async_copy(t_tgt.at[ti], rt, srt).wait()
        pltpu.make_async_copy(t_src.at[si], rs, srs).wait()

    def compute(b, off):
        _, _, ve_v, ef_v, _, _, sve, sef = ivef[b]
        rt, rs, _, _ = rows[b]
        pltpu.make_async_copy(ve_hbm.at[pl.ds(off, CH)], ve_v, sve).wait()
        pltpu.make_async_copy(ef_hbm.at[pl.ds(off, CH)], ef_v, sef).wait()
        wait_rows(b)

        def edge2(k):
            for u in range(1):
                i = k + u
                efe = ef_v[i, :]
                pc = zero16
                vns = []
                for h in range(H):
                    qt, bt = plsc.unpack(rt[i, pl.ds(32 * h, 32)], format=_ILV,
                                         preferred_element_type=jnp.float32)
                    ks, vn = plsc.unpack(rs[i, pl.ds(32 * h, 32)], format=_ILV,
                                         preferred_element_type=jnp.float32)
                    vns.append(vn)
                    lg = jnp.sum(qt * ks + bt * efe)
                    pc = jnp.where(lane == h, lg, pc)
                pv = jnp.exp(pc)
                pay[i, pl.ds(128, 16)] = pv
                for j in range(H // 2):
                    va, vb = plsc.unpack(ve_v[i, pl.ds(32 * j, 32)], format=_ILV,
                                         preferred_element_type=jnp.float32)
                    for p, veh in ((0, va), (1, vb)):
                        h = 2 * j + p
                        pb = _bcast_lane(pv, h)
                        pay[i, pl.ds(16 * h, 16)] = pb * (vns[h] + veh)

        plsc.parallel_loop(0, CH, 1)(edge2)

    def body(g, b, pre1, pre2):
        off = ebase + g * CH
        if pre1:
            wait_idx(off + CH, 1 - b)
            issue_rows(1 - b)
        compute(b, off)
        pltpu.sync_copy(pay, acc.at[ivef[b][1]], add=True)
        if pre2:
            issue_ivef(off + 2 * CH, b)

    issue_ivef(ebase, 0)
    wait_idx(ebase, 0)
    issue_rows(0)
    issue_ivef(ebase + CH, 1)

    def pair(gp, carry):
        body(2 * gp, 0, True, True)
        body(2 * gp + 1, 1, True, True)
        return carry

    lax.fori_loop(0, NG // 2 - 1, pair, None)
    body(NG - 2, 0, True, False)
    body(NG - 1, 1, False, False)
    plsc.subcore_barrier()
    for j in range(nfull):
        pltpu.sync_copy(acc.at[pl.ds(rowbase + j * CH, CH)],
                        out_hbm.at[c, pl.ds(rowbase + j * CH, CH)])
    if rem:
        pltpu.sync_copy(acc.at[pl.ds(rowbase + nfull * CH, rem)],
                        out_hbm.at[c, pl.ds(rowbase + nfull * CH, rem)])


def _pre_call(nf, s_attn, wcat_t, bcat_t, wcat_s):
    grid = N // BN
    return pl.pallas_call(
        _pre_body,
        grid=(grid,),
        in_specs=[
            pl.BlockSpec((BN, D), lambda i: (i, 0)),
            pl.BlockSpec((1, D), lambda i: (0, 0)),
            pl.BlockSpec((D, 2 * D), lambda i: (0, 0)),
            pl.BlockSpec((1, 2 * D), lambda i: (0, 0)),
            pl.BlockSpec((D, 2 * D), lambda i: (0, 0)),
        ],
        out_specs=[
            pl.BlockSpec((BN, 2 * D), lambda i: (i, 0)),
            pl.BlockSpec((BN, 2 * D), lambda i: (i, 0)),
        ],
        out_shape=[
            jax.ShapeDtypeStruct((N, 2 * D), jnp.bfloat16),
            jax.ShapeDtypeStruct((N, 2 * D), jnp.bfloat16),
        ],
    )(nf, s_attn, wcat_t, bcat_t, wcat_s)


def _ve_call(ef, wve, bv):
    grid = E // BE
    return pl.pallas_call(
        _ve_body,
        grid=(grid,),
        in_specs=[
            pl.BlockSpec((BE, DE), lambda i: (i, 0)),
            pl.BlockSpec((DE, D), lambda i: (0, 0)),
            pl.BlockSpec((1, D), lambda i: (0, 0)),
        ],
        out_specs=pl.BlockSpec((BE, D), lambda i: (i, 0)),
        out_shape=jax.ShapeDtypeStruct((E, D), jnp.bfloat16),
    )(ef, wve, bv)


def _post_call(acc, nf, wo, bo, srep, s_ffn, w1, w2):
    grid = N // BN
    return pl.pallas_call(
        _post_body,
        grid=(grid,),
        in_specs=[
            pl.BlockSpec((2, BN, PAY), lambda i: (0, i, 0)),
            pl.BlockSpec((BN, D), lambda i: (i, 0)),
            pl.BlockSpec((D, D), lambda i: (0, 0)),
            pl.BlockSpec((1, D), lambda i: (0, 0)),
            pl.BlockSpec((PAY, D), lambda i: (0, 0)),
            pl.BlockSpec((1, D), lambda i: (0, 0)),
            pl.BlockSpec((D, FFN), lambda i: (0, 0)),
            pl.BlockSpec((FFN, D), lambda i: (0, 0)),
        ],
        out_specs=pl.BlockSpec((BN, D), lambda i: (i, 0)),
        out_shape=jax.ShapeDtypeStruct((N, D), jnp.float32),
    )(acc, nf, wo, bo, srep, s_ffn, w1, w2)


_sc_edge_call = functools.partial(
    pl.kernel,
    out_type=jax.ShapeDtypeStruct((NC, N, PAY), jnp.float32),
    mesh=plsc.VectorSubcoreMesh(core_axis_name="c", subcore_axis_name="s"),
    compiler_params=pltpu.CompilerParams(use_tc_tiling_on_sc=False,
                                         needs_layout_passes=False),
    scratch_types=(
        [pltpu.VMEM((CH,), jnp.int32)] * 4
        + [pltpu.VMEM((CH, D), jnp.bfloat16)] * 2
        + [pltpu.VMEM((CH, DE), jnp.float32)] * 2
        + [pltpu.VMEM((CH, 2 * D), jnp.bfloat16)] * 4
        + [pltpu.VMEM((CH, PAY), jnp.float32)]
        + [pltpu.VMEM_SHARED((N, PAY), jnp.float32)]
        + [pltpu.SemaphoreType.DMA] * 12
    ),
)(_sc_edge_body)


def kernel(node_feats, edge_feats, edge_index, Wq, bq, Wk, bk, Wv, bv,
           Wo, bo, s_attn, s_ffn, W1, W2):
    src = edge_index[0]
    tgt = edge_index[1]
    we = Wk[D:].reshape(DE, H, C)
    wblk = jnp.einsum('jhc,hg->hcgj', we, jnp.eye(H, dtype=jnp.float32))
    wblk = wblk.reshape(H * C, H * DE)
    wq_s = Wq * INV_SQRT_C
    bq_s = bq * INV_SQRT_C
    wcat_t = jnp.concatenate([wq_s, wq_s @ wblk], axis=1)[:, _PERM_T]
    bcat_t = jnp.concatenate([bq_s, bq_s @ wblk])[_PERM_T].reshape(1, 2 * D)
    wcat_s = jnp.concatenate([Wk[:D], Wv[:D]], axis=1)[:, _PERM_T]
    wve = Wv[D:][:, _PERM_V]
    bv_p = bv[_PERM_V].reshape(1, D)
    srep = jnp.concatenate(
        [jnp.zeros((D, D), jnp.float32),
         jnp.kron(jnp.eye(H, dtype=jnp.float32), jnp.ones((1, C), jnp.float32)),
         jnp.zeros((PAY - D - H, D), jnp.float32)], axis=0)

    t_tgt, t_src = _pre_call(node_feats, s_attn.reshape(1, D),
                             wcat_t, bcat_t, wcat_s)
    ve = _ve_call(edge_feats, wve, bv_p)
    acc = _sc_edge_call(t_tgt, t_src, ve, edge_feats, src, tgt)
    out = _post_call(acc, node_feats, Wo, bo.reshape(1, D), srep,
                     s_ffn.reshape(1, D), W1, W2)
    return out


# speedup vs baseline: 3.1426x; 1.0002x over previous
import functools
import math

import jax
import jax.numpy as jnp
import numpy as np
from jax import lax
from jax.experimental import pallas as pl
from jax.experimental.pallas import tpu as pltpu
from jax.experimental.pallas import tpu_sc as plsc

N = 10000
E = 320000
D = 128
DE = 16
H = 8
C = 16
FFN = 512
EPS = 1e-8

PAY = 144
NC, NS = 2, 16
NW = NC * NS
EPT = E // NW
CH = 40
NG = EPT // CH
ROWS_PT = N // NS
SQRT_D = math.sqrt(D)
INV_SQRT_C = 1.0 / math.sqrt(C)

BN = 400
BE = 3200

_ILV = plsc.PackFormat.INTERLEAVED

_PERM_T = np.empty(2 * D, np.int32)
for _h in range(H):
    for _k in range(C):
        _PERM_T[32 * _h + 2 * _k] = 16 * _h + _k
        _PERM_T[32 * _h + 2 * _k + 1] = D + 16 * _h + _k
_PERM_V = np.empty(D, np.int32)
for _j in range(H // 2):
    for _k in range(C):
        _PERM_V[32 * _j + 2 * _k] = 32 * _j + _k
        _PERM_V[32 * _j + 2 * _k + 1] = 32 * _j + 16 + _k


def _bcast_lane(v, h):
    idx = jnp.full((16,), h, jnp.int32)
    return v.at[idx].get(mode="promise_in_bounds")


def _pre_body(nf, s_attn, wcat_t, bcat_t, wcat_s, t_tgt, t_src):
    x = nf[...]
    nrm = jnp.sqrt(jnp.sum(x * x, axis=1, keepdims=True))
    h = s_attn[...] * x / (nrm / SQRT_D + EPS)
    t_tgt[...] = (jnp.dot(h, wcat_t[...], preferred_element_type=jnp.float32)
                  + bcat_t[...]).astype(jnp.bfloat16)
    t_src[...] = jnp.dot(h, wcat_s[...],
                         preferred_element_type=jnp.float32).astype(jnp.bfloat16)


def _ve_body(ef, wve, bv, ve):
    ve[...] = (jnp.dot(ef[...], wve[...], preferred_element_type=jnp.float32)
               + bv[...]).astype(jnp.bfloat16)


def _post_body(acc, nf, wo, bo, srep, s_ffn, w1, w2, out):
    a = acc[0] + acc[1]
    arep = jnp.dot(a, srep[...], preferred_element_type=jnp.float32)
    attn = a[:, :D] * (1.0 / (arep + 1e-16))
    y = jnp.dot(attn, wo[...], preferred_element_type=jnp.float32) + bo[...]
    x1 = nf[...] + y
    nrm = jnp.sqrt(jnp.sum(x1 * x1, axis=1, keepdims=True))
    h2 = s_ffn[...] * x1 / (nrm / SQRT_D + EPS)
    g = jax.nn.gelu(jnp.dot(h2, w1[...], preferred_element_type=jnp.float32))
    out[...] = x1 + jnp.dot(g, w2[...], preferred_element_type=jnp.float32)


def _sc_edge_body(t_tgt, t_src, ve_hbm, ef_hbm, src_hbm, tgt_hbm, out_hbm,
                  sidx0, sidx1, tidx0, tidx1, ve0, ve1, ef0, ef1,
                  rt0, rt1, rs0, rs1, pay, acc,
                  s_si0, s_si1, s_ti0, s_ti1, s_ve0, s_ve1, s_ef0, s_ef1,
                  s_rt0, s_rt1, s_rs0, s_rs1):
    c = lax.axis_index("c")
    s = lax.axis_index("s")
    wid = c * NS + s
    zero16 = jnp.zeros((16,), jnp.float32)
    lane = lax.iota(jnp.int32, 16)
    ivef = ((sidx0, tidx0, ve0, ef0, s_si0, s_ti0, s_ve0, s_ef0),
            (sidx1, tidx1, ve1, ef1, s_si1, s_ti1, s_ve1, s_ef1))
    rows = ((rt0, rs0, s_rt0, s_rs0), (rt1, rs1, s_rt1, s_rs1))

    def zrow(i, carry):
        for j in range(PAY // 16):
            pay[i, pl.ds(j * 16, 16)] = zero16
        return carry

    lax.fori_loop(0, CH, zrow, None)
    rowbase = s * ROWS_PT
    nfull = ROWS_PT // CH
    rem = ROWS_PT - nfull * CH
    for j in range(nfull):
        pltpu.sync_copy(pay.at[pl.ds(0, CH)],
                        acc.at[pl.ds(rowbase + j * CH, CH)])
    if rem:
        pltpu.sync_copy(pay.at[pl.ds(0, rem)],
                        acc.at[pl.ds(rowbase + nfull * CH, rem)])
    plsc.subcore_barrier()

    ebase = wid * EPT

    def issue_ivef(off, b):
        si, ti, ve, ef, ssi, sti, sve, sef = ivef[b]
        pltpu.async_copy(src_hbm.at[pl.ds(off, CH)], si, ssi)
        pltpu.async_copy(tgt_hbm.at[pl.ds(off, CH)], ti, sti)
        pltpu.async_copy(ve_hbm.at[pl.ds(off, CH)], ve, sve)
        pltpu.async_copy(ef_hbm.at[pl.ds(off, CH)], ef, sef)

    def wait_idx(off, b):
        si, ti, _, _, ssi, sti, _, _ = ivef[b]
        pltpu.make_async_copy(src_hbm.at[pl.ds(off, CH)], si, ssi).wait()
        pltpu.make_async_copy(tgt_hbm.at[pl.ds(off, CH)], ti, sti).wait()

    def issue_rows(b):
        si, ti = ivef[b][0], ivef[b][1]
        rt, rs, srt, srs = rows[b]
        pltpu.async_copy(t_tgt.at[ti], rt, srt)
        pltpu.async_copy(t_src.at[si], rs, srs)

    def wait_rows(b):
        si, ti = ivef[b][0], ivef[b][1]
        rt, rs, srt, srs = rows[b]
        pltpu.make_async_copy(t_tgt.at[ti], rt, srt).wait()
        pltpu.make_async_copy(t_src.at[si], rs, srs).wait()

    def compute(b, off):
        _, _, ve_v, ef_v, _, _, sve, sef = ivef[b]
        rt, rs, _, _ = rows[b]
        pltpu.make_async_copy(ve_hbm.at[pl.ds(off, CH)], ve_v, sve).wait()
        pltpu.make_async_copy(ef_hbm.at[pl.ds(off, CH)], ef_v, sef).wait()
        wait_rows(b)

        def edge2(i):
                efe = ef_v[i, :]
                pc = zero16
                vns = []
                for h in range(H):
                    qt, bt = plsc.unpack(rt[i, pl.ds(32 * h, 32)], format=_ILV,
                                         preferred_element_type=jnp.float32)
                    ks, vn = plsc.unpack(rs[i, pl.ds(32 * h, 32)], format=_ILV,
                                         preferred_element_type=jnp.float32)
                    vns.append(vn)
                    lg = jnp.sum(qt * ks + bt * efe)
                    pc = jnp.where(lane == h, lg, pc)
                pv = jnp.exp(pc)
                pay[i, pl.ds(128, 16)] = pv
                for j in range(H // 2):
                    va, vb = plsc.unpack(ve_v[i, pl.ds(32 * j, 32)], format=_ILV,
                                         preferred_element_type=jnp.float32)
                    for p, veh in ((0, va), (1, vb)):
                        h = 2 * j + p
                        pb = _bcast_lane(pv, h)
                        pay[i, pl.ds(16 * h, 16)] = pb * (vns[h] + veh)

        plsc.parallel_loop(0, CH, 1)(edge2)

    def body(g, b, pre1, pre2):
        off = ebase + g * CH
        if pre1:
            wait_idx(off + CH, 1 - b)
            issue_rows(1 - b)
        compute(b, off)
        pltpu.sync_copy(pay, acc.at[ivef[b][1]], add=True)
        if pre2:
            issue_ivef(off + 2 * CH, b)

    issue_ivef(ebase, 0)
    wait_idx(ebase, 0)
    issue_rows(0)
    issue_ivef(ebase + CH, 1)

    def pair(gp, carry):
        body(2 * gp, 0, True, True)
        body(2 * gp + 1, 1, True, True)
        return carry

    lax.fori_loop(0, NG // 2 - 1, pair, None)
    body(NG - 2, 0, True, False)
    body(NG - 1, 1, False, False)
    plsc.subcore_barrier()
    for j in range(nfull):
        pltpu.sync_copy(acc.at[pl.ds(rowbase + j * CH, CH)],
                        out_hbm.at[c, pl.ds(rowbase + j * CH, CH)])
    if rem:
        pltpu.sync_copy(acc.at[pl.ds(rowbase + nfull * CH, rem)],
                        out_hbm.at[c, pl.ds(rowbase + nfull * CH, rem)])


def _pre_call(nf, s_attn, wcat_t, bcat_t, wcat_s):
    grid = N // BN
    return pl.pallas_call(
        _pre_body,
        grid=(grid,),
        in_specs=[
            pl.BlockSpec((BN, D), lambda i: (i, 0)),
            pl.BlockSpec((1, D), lambda i: (0, 0)),
            pl.BlockSpec((D, 2 * D), lambda i: (0, 0)),
            pl.BlockSpec((1, 2 * D), lambda i: (0, 0)),
            pl.BlockSpec((D, 2 * D), lambda i: (0, 0)),
        ],
        out_specs=[
            pl.BlockSpec((BN, 2 * D), lambda i: (i, 0)),
            pl.BlockSpec((BN, 2 * D), lambda i: (i, 0)),
        ],
        out_shape=[
            jax.ShapeDtypeStruct((N, 2 * D), jnp.bfloat16),
            jax.ShapeDtypeStruct((N, 2 * D), jnp.bfloat16),
        ],
    )(nf, s_attn, wcat_t, bcat_t, wcat_s)


def _ve_call(ef, wve, bv):
    grid = E // BE
    return pl.pallas_call(
        _ve_body,
        grid=(grid,),
        in_specs=[
            pl.BlockSpec((BE, DE), lambda i: (i, 0)),
            pl.BlockSpec((DE, D), lambda i: (0, 0)),
            pl.BlockSpec((1, D), lambda i: (0, 0)),
        ],
        out_specs=pl.BlockSpec((BE, D), lambda i: (i, 0)),
        out_shape=jax.ShapeDtypeStruct((E, D), jnp.bfloat16),
    )(ef, wve, bv)


def _post_call(acc, nf, wo, bo, srep, s_ffn, w1, w2):
    grid = N // BN
    return pl.pallas_call(
        _post_body,
        grid=(grid,),
        in_specs=[
            pl.BlockSpec((2, BN, PAY), lambda i: (0, i, 0)),
            pl.BlockSpec((BN, D), lambda i: (i, 0)),
            pl.BlockSpec((D, D), lambda i: (0, 0)),
            pl.BlockSpec((1, D), lambda i: (0, 0)),
            pl.BlockSpec((PAY, D), lambda i: (0, 0)),
            pl.BlockSpec((1, D), lambda i: (0, 0)),
            pl.BlockSpec((D, FFN), lambda i: (0, 0)),
            pl.BlockSpec((FFN, D), lambda i: (0, 0)),
        ],
        out_specs=pl.BlockSpec((BN, D), lambda i: (i, 0)),
        out_shape=jax.ShapeDtypeStruct((N, D), jnp.float32),
    )(acc, nf, wo, bo, srep, s_ffn, w1, w2)


_sc_edge_call = functools.partial(
    pl.kernel,
    out_type=jax.ShapeDtypeStruct((NC, N, PAY), jnp.float32),
    mesh=plsc.VectorSubcoreMesh(core_axis_name="c", subcore_axis_name="s"),
    compiler_params=pltpu.CompilerParams(use_tc_tiling_on_sc=False,
                                         needs_layout_passes=False),
    scratch_types=(
        [pltpu.VMEM((CH,), jnp.int32)] * 4
        + [pltpu.VMEM((CH, D), jnp.bfloat16)] * 2
        + [pltpu.VMEM((CH, DE), jnp.float32)] * 2
        + [pltpu.VMEM((CH, 2 * D), jnp.bfloat16)] * 4
        + [pltpu.VMEM((CH, PAY), jnp.float32)]
        + [pltpu.VMEM_SHARED((N, PAY), jnp.float32)]
        + [pltpu.SemaphoreType.DMA] * 12
    ),
)(_sc_edge_body)


def kernel(node_feats, edge_feats, edge_index, Wq, bq, Wk, bk, Wv, bv,
           Wo, bo, s_attn, s_ffn, W1, W2):
    src = edge_index[0]
    tgt = edge_index[1]
    we = Wk[D:].reshape(DE, H, C)
    wblk = jnp.einsum('jhc,hg->hcgj', we, jnp.eye(H, dtype=jnp.float32))
    wblk = wblk.reshape(H * C, H * DE)
    wq_s = Wq * INV_SQRT_C
    bq_s = bq * INV_SQRT_C
    wcat_t = jnp.concatenate([wq_s, wq_s @ wblk], axis=1)[:, _PERM_T]
    bcat_t = jnp.concatenate([bq_s, bq_s @ wblk])[_PERM_T].reshape(1, 2 * D)
    wcat_s = jnp.concatenate([Wk[:D], Wv[:D]], axis=1)[:, _PERM_T]
    wve = Wv[D:][:, _PERM_V]
    bv_p = bv[_PERM_V].reshape(1, D)
    srep = jnp.concatenate(
        [jnp.zeros((D, D), jnp.float32),
         jnp.kron(jnp.eye(H, dtype=jnp.float32), jnp.ones((1, C), jnp.float32)),
         jnp.zeros((PAY - D - H, D), jnp.float32)], axis=0)

    t_tgt, t_src = _pre_call(node_feats, s_attn.reshape(1, D),
                             wcat_t, bcat_t, wcat_s)
    ve = _ve_call(edge_feats, wve, bv_p)
    acc = _sc_edge_call(t_tgt, t_src, ve, edge_feats, src, tgt)
    out = _post_call(acc, node_feats, Wo, bo.reshape(1, D), srep,
                     s_ffn.reshape(1, D), W1, W2)
    return out
